# baseline (device time: 191571 ns/iter reference)
import jax
import jax.numpy as jnp
from jax import lax
from jax.experimental import pallas as pl
from jax.experimental.pallas import tpu as pltpu

N_DEV = 4
SQ = 1024
SKV_LOC = 1024
HQ = 8
DH = 128
D = 1024
SCALE = 0.08838834764831843


def kernel(x, Wq, K_ext, V_ext, Wo):
    x2 = x[0]
    K = jnp.transpose(K_ext[0], (1, 0, 2))
    V = jnp.transpose(V_ext[0], (1, 0, 2))

    def body(x_ref, wq_ref, k_ref, v_ref, wo_ref, out_ref,
             comm_ref, lcomm_ref, acc_ref, lacc_ref,
             send_sems, recv_sems, lsend_sems, lrecv_sems):
        my = lax.axis_index("i")
        left = (my + N_DEV - 1) % N_DEV
        right = (my + 1) % N_DEV

        barrier = pltpu.get_barrier_semaphore()
        for nbr in (left, right):
            pl.semaphore_signal(barrier, inc=1, device_id=(nbr,),
                                device_id_type=pl.DeviceIdType.MESH)
        pl.semaphore_wait(barrier, 2)

        qi = lax.broadcasted_iota(jnp.int32, (SQ, SKV_LOC), 0)
        kj = lax.broadcasted_iota(jnp.int32, (SQ, SKV_LOC), 1) + my * SKV_LOC
        mask = (jnp.abs(qi - kj) <= 128) | (kj < 32) | (qi < 32)

        for h in range(HQ):
            hs = pl.ds(h * DH, DH)
            q_h = jnp.dot(x_ref[...], wq_ref[:, hs],
                          preferred_element_type=jnp.float32)
            s = lax.dot_general(q_h, k_ref[h], (((1,), (1,)), ((), ())),
                                preferred_element_type=jnp.float32)
            w = jnp.where(mask, jnp.exp(s * SCALE), 0.0)
            l_h = jnp.sum(w, axis=1, keepdims=True)
            ctx_h = jnp.dot(w, v_ref[h],
                            preferred_element_type=jnp.float32)
            comm_ref[0, :, hs] = ctx_h
            acc_ref[:, hs] = ctx_h
            lcomm_ref[0, :, pl.ds(h, 1)] = l_h
            lacc_ref[:, pl.ds(h, 1)] = l_h

        for hop in range(N_DEV - 1):
            s_slot = hop % 2
            r_slot = (hop + 1) % 2
            rdma = pltpu.make_async_remote_copy(
                src_ref=comm_ref.at[s_slot],
                dst_ref=comm_ref.at[r_slot],
                send_sem=send_sems.at[s_slot],
                recv_sem=recv_sems.at[r_slot],
                device_id=(right,),
                device_id_type=pl.DeviceIdType.MESH,
            )
            lrdma = pltpu.make_async_remote_copy(
                src_ref=lcomm_ref.at[s_slot],
                dst_ref=lcomm_ref.at[r_slot],
                send_sem=lsend_sems.at[s_slot],
                recv_sem=lrecv_sems.at[r_slot],
                device_id=(right,),
                device_id_type=pl.DeviceIdType.MESH,
            )
            rdma.start()
            lrdma.start()
            rdma.wait()
            lrdma.wait()
            acc_ref[...] = acc_ref[...] + comm_ref[r_slot]
            lacc_ref[...] = lacc_ref[...] + lcomm_ref[r_slot]

        for h in range(HQ):
            hs = pl.ds(h * DH, DH)
            acc_ref[:, hs] = acc_ref[:, hs] / lacc_ref[:, pl.ds(h, 1)]
        out_ref[...] = jnp.dot(acc_ref[...], wo_ref[...],
                               preferred_element_type=jnp.float32)

    out = pl.pallas_call(
        body,
        out_shape=jax.ShapeDtypeStruct((SQ, D), jnp.float32),
        in_specs=[pl.BlockSpec(memory_space=pltpu.VMEM)] * 5,
        out_specs=pl.BlockSpec(memory_space=pltpu.VMEM),
        scratch_shapes=[
            pltpu.VMEM((2, SQ, D), jnp.float32),
            pltpu.VMEM((2, SQ, HQ), jnp.float32),
            pltpu.VMEM((SQ, D), jnp.float32),
            pltpu.VMEM((SQ, HQ), jnp.float32),
            pltpu.SemaphoreType.DMA((2,)),
            pltpu.SemaphoreType.DMA((2,)),
            pltpu.SemaphoreType.DMA((2,)),
            pltpu.SemaphoreType.DMA((2,)),
        ],
        compiler_params=pltpu.CompilerParams(collective_id=0),
    )(x2, Wq, K, V, Wo)
    return out[None]


# device time: 124153 ns/iter; 1.5430x vs baseline; 1.5430x over previous
import jax
import jax.numpy as jnp
from jax import lax
from jax.experimental import pallas as pl
from jax.experimental.pallas import tpu as pltpu

N_DEV = 4
SQ = 1024
SKV_LOC = 1024
HQ = 8
DH = 128
D = 1024
SCALE = 0.08838834764831843


def kernel(x, Wq, K_ext, V_ext, Wo):
    x2 = x[0]
    K = jnp.transpose(K_ext[0], (1, 0, 2))
    V = jnp.transpose(V_ext[0], (1, 0, 2))

    def body(x_ref, wq_ref, k_ref, v_ref, wo_ref, out_ref,
             comm_ref, lcomm_ref, acc_ref, lacc_ref,
             send_sems, recv_sems, lsend_sems, lrecv_sems):
        my = lax.axis_index("i")
        left = (my + N_DEV - 1) % N_DEV
        right = (my + 1) % N_DEV

        barrier = pltpu.get_barrier_semaphore()
        for nbr in (left, right):
            pl.semaphore_signal(barrier, inc=1, device_id=(nbr,),
                                device_id_type=pl.DeviceIdType.MESH)
        pl.semaphore_wait(barrier, 2)

        qi = lax.broadcasted_iota(jnp.int32, (SQ, SKV_LOC), 0)
        kj = lax.broadcasted_iota(jnp.int32, (SQ, SKV_LOC), 1) + my * SKV_LOC
        mask = (jnp.abs(qi - kj) <= 128) | (kj < 32) | (qi < 32)

        for h in range(HQ):
            hs = pl.ds(h * DH, DH)
            q_h = jnp.dot(x_ref[...], wq_ref[:, hs],
                          preferred_element_type=jnp.float32)
            s = lax.dot_general(q_h, k_ref[h], (((1,), (1,)), ((), ())),
                                preferred_element_type=jnp.float32)
            w = jnp.where(mask, jnp.exp(s * SCALE), 0.0)
            l_h = jnp.sum(w, axis=1, keepdims=True)
            ctx_h = jnp.dot(w, v_ref[h],
                            preferred_element_type=jnp.float32)
            comm_ref[0, :, hs] = ctx_h.astype(jnp.bfloat16)
            acc_ref[:, hs] = ctx_h
            lcomm_ref[0, :, pl.ds(h, 1)] = l_h
            lacc_ref[:, pl.ds(h, 1)] = l_h

        for hop in range(N_DEV - 1):
            s_slot = hop % 2
            r_slot = (hop + 1) % 2
            rdma = pltpu.make_async_remote_copy(
                src_ref=comm_ref.at[s_slot],
                dst_ref=comm_ref.at[r_slot],
                send_sem=send_sems.at[s_slot],
                recv_sem=recv_sems.at[r_slot],
                device_id=(right,),
                device_id_type=pl.DeviceIdType.MESH,
            )
            lrdma = pltpu.make_async_remote_copy(
                src_ref=lcomm_ref.at[s_slot],
                dst_ref=lcomm_ref.at[r_slot],
                send_sem=lsend_sems.at[s_slot],
                recv_sem=lrecv_sems.at[r_slot],
                device_id=(right,),
                device_id_type=pl.DeviceIdType.MESH,
            )
            rdma.start()
            lrdma.start()
            rdma.wait()
            lrdma.wait()
            acc_ref[...] = acc_ref[...] + comm_ref[r_slot].astype(jnp.float32)
            lacc_ref[...] = lacc_ref[...] + lcomm_ref[r_slot]

        for h in range(HQ):
            hs = pl.ds(h * DH, DH)
            acc_ref[:, hs] = acc_ref[:, hs] / lacc_ref[:, pl.ds(h, 1)]
        out_ref[...] = jnp.dot(acc_ref[...], wo_ref[...],
                               preferred_element_type=jnp.float32)

    out = pl.pallas_call(
        body,
        out_shape=jax.ShapeDtypeStruct((SQ, D), jnp.float32),
        in_specs=[pl.BlockSpec(memory_space=pltpu.VMEM)] * 5,
        out_specs=pl.BlockSpec(memory_space=pltpu.VMEM),
        scratch_shapes=[
            pltpu.VMEM((2, SQ, D), jnp.bfloat16),
            pltpu.VMEM((2, SQ, HQ), jnp.float32),
            pltpu.VMEM((SQ, D), jnp.float32),
            pltpu.VMEM((SQ, HQ), jnp.float32),
            pltpu.SemaphoreType.DMA((2,)),
            pltpu.SemaphoreType.DMA((2,)),
            pltpu.SemaphoreType.DMA((2,)),
            pltpu.SemaphoreType.DMA((2,)),
        ],
        compiler_params=pltpu.CompilerParams(collective_id=0),
    )(x2, Wq, K, V, Wo)
    return out[None]


# device time: 53346 ns/iter; 3.5911x vs baseline; 2.3273x over previous
import jax
import jax.numpy as jnp
from jax import lax
from jax.experimental import pallas as pl
from jax.experimental.pallas import tpu as pltpu

N_DEV = 4
SQ = 1024
SKV_LOC = 1024
HQ = 8
DH = 128
D = 1024
SCALE = 0.08838834764831843

F32 = jnp.float32
BF16 = jnp.bfloat16
MESH = pl.DeviceIdType.MESH


def kernel(x, Wq, K_ext, V_ext, Wo):
    x2 = x[0]
    K = jnp.transpose(K_ext[0], (1, 0, 2))
    V = jnp.transpose(V_ext[0], (1, 0, 2))

    def body(x_ref, wq_ref, k_ref, v_ref, wo_ref, out_ref,
             p0_ref, l0_ref, edge_ref, glob_ref, acc_ref, lt_ref,
             sp0, rp0, s_small, r_small):
        my = lax.axis_index("i")

        barrier = pltpu.get_barrier_semaphore()
        for t in range(N_DEV):
            @pl.when(my != t)
            def _():
                pl.semaphore_signal(barrier, inc=1, device_id=(t,),
                                    device_id_type=MESH)
        pl.semaphore_wait(barrier, N_DEV - 1)

        def copy(src, dst, ssem, rsem, dev):
            return pltpu.make_async_remote_copy(
                src_ref=src, dst_ref=dst, send_sem=ssem, recv_sem=rsem,
                device_id=(dev,), device_id_type=MESH)

        def send_small(buf, base, targets, ridx):
            for j, t in enumerate(targets):
                copy(buf, buf, s_small.at[base + j], r_small.at[ridx],
                     t).start()

        def wait_small(buf, ridx):
            copy(buf, buf, s_small.at[0], r_small.at[ridx], 0).wait_recv()

        def glob_partial(h):
            qg = jnp.dot(x_ref[pl.ds(0, 32), :], wq_ref[:, pl.ds(h * DH, DH)],
                         preferred_element_type=F32)
            sg = lax.dot_general(qg, k_ref[h], (((1,), (1,)), ((), ())),
                                 preferred_element_type=F32)
            wg = jnp.exp(sg * SCALE)
            return (jnp.dot(wg, v_ref[h], preferred_element_type=F32),
                    jnp.sum(wg, axis=1, keepdims=True))

        @pl.when(my == 0)
        def _dev0():
            qi = lax.broadcasted_iota(jnp.int32, (SQ, SKV_LOC), 0)
            kj = lax.broadcasted_iota(jnp.int32, (SQ, SKV_LOC), 1)
            mask = (jnp.abs(qi - kj) <= 128) | (kj < 32) | (qi < 32)
            for h in range(HQ):
                q_h = jnp.dot(x_ref[...], wq_ref[:, pl.ds(h * DH, DH)],
                              preferred_element_type=F32)
                s = lax.dot_general(q_h, k_ref[h], (((1,), (1,)), ((), ())),
                                    preferred_element_type=F32)
                w = jnp.where(mask, jnp.exp(s * SCALE), 0.0)
                l0_ref[:, pl.ds(h, 1)] = jnp.sum(w, axis=1, keepdims=True)
                ctx = jnp.dot(w, v_ref[h],
                              preferred_element_type=F32).astype(BF16)
                p0_ref[h, 0] = ctx[0:512]
                p0_ref[h, 1] = ctx[512:1024]
                for j, t in enumerate((1, 3)):
                    copy(p0_ref.at[h], p0_ref.at[h], sp0.at[j, h],
                         rp0.at[0, h], t).start()
            send_small(l0_ref, 0, (1, 2, 3), 0)
            for ridx in (1, 2, 3, 4):
                wait_small(edge_ref if ridx == 1 else
                           glob_ref.at[ridx - 2], ridx)

        @pl.when(my == 1)
        def _dev1():
            qi = lax.broadcasted_iota(jnp.int32, (128, SKV_LOC), 0) + 896
            kj = lax.broadcasted_iota(jnp.int32, (128, SKV_LOC), 1) + SKV_LOC
            mask_e = jnp.abs(qi - kj) <= 128
            for h in range(HQ):
                ctx_g, l_g = glob_partial(h)
                glob_ref[0, h] = ctx_g.astype(BF16)
                glob_ref[0, 8, :, pl.ds(h, 1)] = l_g.astype(BF16)
                qe = jnp.dot(x_ref[pl.ds(896, 128), :],
                             wq_ref[:, pl.ds(h * DH, DH)],
                             preferred_element_type=F32)
                se = lax.dot_general(qe, k_ref[h], (((1,), (1,)), ((), ())),
                                     preferred_element_type=F32)
                we = jnp.where(mask_e, jnp.exp(se * SCALE), 0.0)
                edge_ref[h] = jnp.dot(we, v_ref[h],
                                      preferred_element_type=F32).astype(BF16)
                edge_ref[8, :, pl.ds(h, 1)] = jnp.sum(
                    we, axis=1, keepdims=True).astype(BF16)
            send_small(edge_ref, 0, (0, 2, 3), 1)
            send_small(glob_ref.at[0], 3, (0, 2, 3), 2)
            for h in range(HQ):
                copy(p0_ref.at[h], p0_ref.at[h], sp0.at[1, h],
                     rp0.at[0, h], 0).wait_recv()
                copy(p0_ref.at[h, 0], p0_ref.at[h, 0], sp0.at[0, h],
                     rp0.at[0, h], 2).start()
            wait_small(l0_ref, 0)
            wait_small(glob_ref.at[1], 3)
            wait_small(glob_ref.at[2], 4)

        @pl.when(my == 2)
        def _dev2():
            for h in range(HQ):
                ctx_g, l_g = glob_partial(h)
                glob_ref[1, h] = ctx_g.astype(BF16)
                glob_ref[1, 8, :, pl.ds(h, 1)] = l_g.astype(BF16)
            send_small(glob_ref.at[1], 0, (0, 1, 3), 3)
            for h in range(HQ):
                copy(p0_ref.at[h, 0], p0_ref.at[h, 0], sp0.at[1, h],
                     rp0.at[0, h], 0).wait_recv()
                copy(p0_ref.at[h, 1], p0_ref.at[h, 1], sp0.at[1, h],
                     rp0.at[1, h], 0).wait_recv()
            wait_small(l0_ref, 0)
            wait_small(edge_ref, 1)
            wait_small(glob_ref.at[0], 2)
            wait_small(glob_ref.at[2], 4)

        @pl.when(my == 3)
        def _dev3():
            for h in range(HQ):
                ctx_g, l_g = glob_partial(h)
                glob_ref[2, h] = ctx_g.astype(BF16)
                glob_ref[2, 8, :, pl.ds(h, 1)] = l_g.astype(BF16)
            send_small(glob_ref.at[2], 0, (0, 1, 2), 4)
            for h in range(HQ):
                copy(p0_ref.at[h], p0_ref.at[h], sp0.at[1, h],
                     rp0.at[0, h], 0).wait_recv()
                copy(p0_ref.at[h, 1], p0_ref.at[h, 1], sp0.at[0, h],
                     rp0.at[1, h], 2).start()
            wait_small(l0_ref, 0)
            wait_small(edge_ref, 1)
            wait_small(glob_ref.at[0], 2)
            wait_small(glob_ref.at[1], 3)

        lt_ref[...] = l0_ref[...]
        lt_ref[pl.ds(896, 128), :] = (lt_ref[pl.ds(896, 128), :]
                                      + edge_ref[8, :, 0:8].astype(F32))
        lt_ref[pl.ds(0, 32), :] = (lt_ref[pl.ds(0, 32), :]
                                   + glob_ref[0, 8, :, 0:8].astype(F32)
                                   + glob_ref[1, 8, :, 0:8].astype(F32)
                                   + glob_ref[2, 8, :, 0:8].astype(F32))
        for h in range(HQ):
            hs = pl.ds(h * DH, DH)
            acc_ref[pl.ds(0, 512), hs] = p0_ref[h, 0].astype(F32)
            acc_ref[pl.ds(512, 512), hs] = p0_ref[h, 1].astype(F32)
            acc_ref[pl.ds(896, 128), hs] = (acc_ref[pl.ds(896, 128), hs]
                                            + edge_ref[h].astype(F32))
            acc_ref[pl.ds(0, 32), hs] = (acc_ref[pl.ds(0, 32), hs]
                                         + glob_ref[0, h].astype(F32)
                                         + glob_ref[1, h].astype(F32)
                                         + glob_ref[2, h].astype(F32))
        for h in range(HQ):
            hs = pl.ds(h * DH, DH)
            acc_ref[:, hs] = acc_ref[:, hs] / lt_ref[:, pl.ds(h, 1)]
        out_ref[...] = jnp.dot(acc_ref[...], wo_ref[...],
                               preferred_element_type=F32)

        @pl.when(my == 0)
        def _():
            for h in range(HQ):
                for j in range(2):
                    copy(p0_ref.at[h], p0_ref.at[h], sp0.at[j, h],
                         rp0.at[0, h], 1).wait_send()
            for j in range(3):
                copy(l0_ref, l0_ref, s_small.at[j], r_small.at[0],
                     1).wait_send()

        @pl.when(my == 1)
        def _():
            for h in range(HQ):
                copy(p0_ref.at[h, 0], p0_ref.at[h, 0], sp0.at[0, h],
                     rp0.at[0, h], 2).wait_send()
            for j in range(3):
                copy(edge_ref, edge_ref, s_small.at[j], r_small.at[1],
                     0).wait_send()
                copy(glob_ref.at[0], glob_ref.at[0], s_small.at[3 + j],
                     r_small.at[2], 0).wait_send()

        @pl.when(my == 2)
        def _():
            for j in range(3):
                copy(glob_ref.at[1], glob_ref.at[1], s_small.at[j],
                     r_small.at[3], 0).wait_send()

        @pl.when(my == 3)
        def _():
            for h in range(HQ):
                copy(p0_ref.at[h, 1], p0_ref.at[h, 1], sp0.at[0, h],
                     rp0.at[1, h], 2).wait_send()
            for j in range(3):
                copy(glob_ref.at[2], glob_ref.at[2], s_small.at[j],
                     r_small.at[4], 0).wait_send()

    out = pl.pallas_call(
        body,
        out_shape=jax.ShapeDtypeStruct((SQ, D), F32),
        in_specs=[pl.BlockSpec(memory_space=pltpu.VMEM)] * 5,
        out_specs=pl.BlockSpec(memory_space=pltpu.VMEM),
        scratch_shapes=[
            pltpu.VMEM((HQ, 2, 512, DH), BF16),
            pltpu.VMEM((SQ, HQ), F32),
            pltpu.VMEM((HQ + 1, 128, DH), BF16),
            pltpu.VMEM((3, HQ + 1, 32, DH), BF16),
            pltpu.VMEM((SQ, D), F32),
            pltpu.VMEM((SQ, HQ), F32),
            pltpu.SemaphoreType.DMA((2, HQ)),
            pltpu.SemaphoreType.DMA((2, HQ)),
            pltpu.SemaphoreType.DMA((6,)),
            pltpu.SemaphoreType.DMA((5,)),
        ],
        compiler_params=pltpu.CompilerParams(collective_id=0),
    )(x2, Wq, K, V, Wo)
    return out[None]
